# baseline (speedup 1.0000x reference)
import functools

import jax
import jax.numpy as jnp
from jax import lax
from jax.experimental import pallas as pl
from jax.experimental.pallas import tpu as pltpu
from jax.experimental.pallas import tpu_sc as plsc

N_NODES = 100000
NUM_GRAPHS = 64
NR = 128
RANGE = 784
SLAB = RANGE + 8
K = 128
GP = NR * RANGE
ZROW = N_NODES


def _seg_kernel(g_h, srcp_h, didxp_h, meta_h, zeros_h, out_h,
                slab, rows, sidx, didx, meta_v, sem):
    c = lax.axis_index("c")
    s = lax.axis_index("s")
    wid = s * 2 + c
    pltpu.sync_copy(meta_h, meta_v)
    mrow = meta_v[wid, :]
    for j in range(4):
        r = wid * 4 + j
        pstart = mrow[j]
        ntr = mrow[4 + j]
        pltpu.sync_copy(zeros_h, slab.at[pl.ds(pl.multiple_of(s * SLAB, 8), SLAB)])

        def body(t, carry):
            off = pl.multiple_of(pstart + t * K, K)
            pltpu.sync_copy(srcp_h.at[pl.ds(off, K)], sidx)
            pltpu.sync_copy(didxp_h.at[pl.ds(off, K)], didx)
            pltpu.async_copy(g_h.at[sidx], rows, sem).wait()
            pltpu.sync_copy(rows, slab.at[didx], add=True)
            return carry

        lax.fori_loop(0, ntr, body, 0)
        pltpu.sync_copy(slab.at[pl.ds(pl.multiple_of(s * SLAB, 8), RANGE)],
                        out_h.at[pl.ds(pl.multiple_of(r * RANGE, 8), RANGE)])


def _seg_sum(g_pad, srcp, didxp, meta, zeros):
    mesh = plsc.VectorSubcoreMesh(core_axis_name="c", subcore_axis_name="s")
    fn = functools.partial(
        pl.kernel,
        mesh=mesh,
        out_type=jax.ShapeDtypeStruct((GP, 128), jnp.float32),
        scratch_types=[
            pltpu.VMEM_SHARED((16 * SLAB, 128), jnp.float32),
            pltpu.VMEM((K, 128), jnp.float32),
            pltpu.VMEM((K,), jnp.int32),
            pltpu.VMEM((K,), jnp.int32),
            pltpu.VMEM((32, 16), jnp.int32),
            pltpu.SemaphoreType.DMA,
        ],
    )(_seg_kernel)
    return fn(g_pad, srcp, didxp, meta, zeros)


def _head_kernel(xg_ref, xt_ref, w1_ref, b1_ref, w2_ref, b2_ref, wo_ref, bo_ref, out_ref):
    xc = jnp.concatenate([xg_ref[...], xt_ref[...]], axis=1)
    h1 = jnp.maximum(xc @ w1_ref[...] + b1_ref[...], 0.0)
    h2 = jnp.maximum(h1 @ w2_ref[...] + b2_ref[...], 0.0)
    out_ref[...] = jnp.sum(h2 * wo_ref[...].T, axis=1, keepdims=True) + bo_ref[...]


def _head(xg, xt, p):
    return pl.pallas_call(
        _head_kernel,
        out_shape=jax.ShapeDtypeStruct((xg.shape[0], 1), jnp.float32),
    )(xg, xt, p['W_fc1'], p['b_fc1'], p['W_fc2'], p['b_fc2'], p['W_out'], p['b_out'])


def _lstm(x_seq, Wi, Wh, b, reverse):
    B = x_seq.shape[0]
    H = Wh.shape[0]
    def step(carry, xt):
        h, c = carry
        g = xt @ Wi + h @ Wh + b
        i, f, gg, o = jnp.split(g, 4, axis=-1)
        i = jax.nn.sigmoid(i)
        f = jax.nn.sigmoid(f)
        gg = jnp.tanh(gg)
        o = jax.nn.sigmoid(o)
        c = f * c + i * gg
        h = o * jnp.tanh(c)
        return (h, c), h
    xs = jnp.swapaxes(x_seq, 0, 1)
    init = (jnp.zeros((B, H), x_seq.dtype), jnp.zeros((B, H), x_seq.dtype))
    _, hs = jax.lax.scan(step, init, xs, reverse=reverse)
    return jnp.swapaxes(hs, 0, 1)


def kernel(x, edge_index, batch, target, params):
    p = params
    N = x.shape[0]
    E = edge_index.shape[1]
    src = edge_index[0]
    dst = edge_index[1]
    deg = jnp.bincount(edge_index.reshape(-1), length=N)
    deg_f = deg.astype(jnp.float32)

    rid = (dst // RANGE).astype(jnp.int32)
    perm = jnp.argsort(rid)
    src_b = src[perm].astype(jnp.int32)
    dst_b = dst[perm].astype(jnp.int32)
    rid_b = rid[perm]
    counts = jnp.bincount(rid, length=NR).astype(jnp.int32)
    start = jnp.concatenate([jnp.zeros((1,), jnp.int32),
                             jnp.cumsum(counts).astype(jnp.int32)])[:NR]
    ntrips = (counts + (K - 1)) // K
    pcounts = ntrips * K
    pstart = jnp.concatenate([jnp.zeros((1,), jnp.int32),
                              jnp.cumsum(pcounts).astype(jnp.int32)])[:NR]
    ppos = pstart[rid_b] + (jnp.arange(E, dtype=jnp.int32) - start[rid_b])
    PE = E + NR * K
    srcp = jnp.full((PE,), ZROW, jnp.int32).at[ppos].set(src_b)
    cum_pc = jnp.cumsum(pcounts).astype(jnp.int32)
    slot_rid = jnp.clip(
        jnp.searchsorted(cum_pc, jnp.arange(PE, dtype=jnp.int32), side='right'),
        0, NR - 1).astype(jnp.int32)
    didxp = ((slot_rid // 8) * SLAB + RANGE).astype(jnp.int32)
    didxp = didxp.at[ppos].set((rid_b // 8) * SLAB + dst_b - rid_b * RANGE)
    meta = jnp.concatenate([
        pstart.reshape(32, 4), ntrips.reshape(32, 4),
        jnp.zeros((32, 8), jnp.int32)], axis=1).astype(jnp.int32)
    zeros = jnp.zeros((SLAB, 128), jnp.float32)

    def gin(h, l):
        g_pad = jnp.concatenate([
            (4.0 - deg_f)[:, None] * h,
            h,
            jnp.zeros((N, 64), jnp.float32),
        ], axis=1)
        g_pad = jnp.concatenate(
            [g_pad, jnp.zeros((GP - N, 128), jnp.float32)], axis=0)
        out2 = _seg_sum(g_pad, srcp, didxp, meta, zeros)
        agg = out2[:N, :32] - deg_f[:, None] * out2[:N, 32:64]
        o = h + agg
        return jnp.maximum(o @ p['W1_%d' % l] + p['b1_%d' % l], 0.0) @ p['W2_%d' % l] + p['b2_%d' % l]

    def bn(h, l):
        m = jnp.mean(h, axis=0)
        v = jnp.var(h, axis=0)
        return (h - m) / jnp.sqrt(v + 1e-5) * p['g_%d' % l] + p['be_%d' % l]

    h = x @ p['W_ft'] + p['b_ft']
    h1 = bn(jax.nn.relu(gin(h, 1)), 1)
    h2 = bn(jax.nn.relu(gin(h1, 2)) + h1, 2)
    h3 = bn(jax.nn.relu(gin(h2, 3)) + h2, 3)
    h4 = bn(jax.nn.relu(gin(h3, 4)) + h3, 4)
    h5 = bn(jax.nn.relu(gin(h4, 5)) + h4, 5)
    xg = jax.ops.segment_sum(h5, batch, num_segments=NUM_GRAPHS)
    xg = jax.nn.relu(xg @ p['W_fc1xd'] + p['b_fc1xd'])

    emb = p['emb'][target]
    hf = _lstm(emb, p['Wi_f'], p['Wh_f'], p['b_f'], False)
    hb = _lstm(emb, p['Wi_b'], p['Wh_b'], p['b_b'], True)
    lo = jnp.concatenate([hf, hb], axis=-1)
    aw = jax.nn.softmax(lo @ p['W_attn'] + p['b_attn'], axis=1)
    ctx = jnp.sum(aw * lo, axis=1)
    xt = jax.nn.relu(ctx @ p['W_fc1xt'] + p['b_fc1xt'])

    return _head(xg, xt, p)
